# baseline (speedup 1.0000x reference)
import functools

import jax
import jax.numpy as jnp
from jax import lax
from jax.experimental import pallas as pl
from jax.experimental.pallas import tpu as pltpu
from jax.experimental.pallas import tpu_sc as plsc

DIM = 64
BATCH = 16384
HIST = 200
NC = 2
NS = 16
NW = NC * NS
ROWS_W = BATCH // NW
NB = 4
NCHUNK = ROWS_W // NB
SPLITS = ((0, 128), (128, HIST - 128))


def _make_sc_gather():
  mesh = plsc.VectorSubcoreMesh(core_axis_name="c", subcore_axis_name="s")

  @functools.partial(
      pl.kernel,
      mesh=mesh,
      out_type=jax.ShapeDtypeStruct((BATCH, HIST, DIM), jnp.float32),
      compiler_params=pltpu.CompilerParams(use_tc_tiling_on_sc=False),
      scratch_types=[
          pltpu.VMEM((2, NB, HIST), jnp.int32),
          pltpu.VMEM((2, NB, HIST, DIM), jnp.float32),
          pltpu.SemaphoreType.DMA,
          pltpu.SemaphoreType.DMA,
          pltpu.SemaphoreType.DMA,
          pltpu.SemaphoreType.DMA,
          pltpu.SemaphoreType.DMA,
          pltpu.SemaphoreType.DMA,
      ],
  )
  def sc_gather(x_hbm, table_hbm, out_hbm, idx_v, rows_v,
                is0, is1, gs0, gs1, ss0, ss1):
    wid = lax.axis_index("s") * NC + lax.axis_index("c")
    row0 = wid * ROWS_W
    isem = (is0, is1)
    gsem = (gs0, gs1)
    ssem = (ss0, ss1)

    def load_idx(c, b):
      pltpu.async_copy(x_hbm.at[pl.ds(row0 + c * NB, NB)],
                       idx_v.at[b], isem[b])

    def wait_idx(b):
      pltpu.make_async_copy(x_hbm.at[pl.ds(0, NB)], idx_v.at[b],
                            isem[b]).wait()

    def gather(b):
      handles = []
      for i in range(NB):
        for off, n in SPLITS:
          handles.append(pltpu.async_copy(
              table_hbm.at[idx_v.at[b, i, pl.ds(off, n)]],
              rows_v.at[b, i, pl.ds(off, n)], gsem[b]))
      for h in handles:
        h.wait()

    def store(c, b):
      pltpu.async_copy(rows_v.at[b], out_hbm.at[pl.ds(row0 + c * NB, NB)],
                       ssem[b])

    def wait_store(b):
      pltpu.make_async_copy(rows_v.at[b], out_hbm.at[pl.ds(0, NB)],
                            ssem[b]).wait()

    load_idx(0, 0)
    load_idx(1, 1)
    for b in range(2):
      wait_idx(b)
      gather(b)
      store(b, b)
      load_idx(b + 2, b)

    def body(g, carry):
      for b in range(2):
        c = 2 + g * 2 + b
        wait_idx(b)
        wait_store(b)
        gather(b)
        store(c, b)
        load_idx(jnp.minimum(c + 2, NCHUNK - 1), b)
      return carry

    lax.fori_loop(0, (NCHUNK - 2) // 2, body, 0, unroll=False)

    for b in range(2):
      wait_store(b)
      wait_idx(b)

  return sc_gather


_sc_gather = _make_sc_gather()


@jax.jit
def kernel(x, table):
  return _sc_gather(x.astype(jnp.int32), table)


# speedup vs baseline: 1.2990x; 1.2990x over previous
import functools

import jax
import jax.numpy as jnp
from jax import lax
from jax.experimental import pallas as pl
from jax.experimental.pallas import tpu as pltpu
from jax.experimental.pallas import tpu_sc as plsc

DIM = 64
PAD = 128
BATCH = 16384
HIST = 200
NC = 2
NS = 16
NW = NC * NS
ROWS_W = BATCH // NW
IB = 8
NB = 2
NSUBC = IB // NB
NIDX = ROWS_W // IB
SPLITS = ((0, 128), (128, HIST - 128))


def _make_sc_gather():
  mesh = plsc.VectorSubcoreMesh(core_axis_name="c", subcore_axis_name="s")

  @functools.partial(
      pl.kernel,
      mesh=mesh,
      out_type=jax.ShapeDtypeStruct((BATCH, HIST, PAD), jnp.float32),
      compiler_params=pltpu.CompilerParams(use_tc_tiling_on_sc=True),
      scratch_types=[
          pltpu.VMEM((2, IB, HIST), jnp.int32),
          pltpu.VMEM((2, NB, HIST, PAD), jnp.float32),
          pltpu.SemaphoreType.DMA,
          pltpu.SemaphoreType.DMA,
          pltpu.SemaphoreType.DMA,
          pltpu.SemaphoreType.DMA,
          pltpu.SemaphoreType.DMA,
          pltpu.SemaphoreType.DMA,
      ],
  )
  def sc_gather(x_hbm, table_hbm, out_hbm, idx_v, pair_v,
                is0, is1, gs0, gs1, ss0, ss1):
    wid = lax.axis_index("s") * NC + lax.axis_index("c")
    row0 = wid * ROWS_W
    isem = (is0, is1)
    gsem = (gs0, gs1)
    ssem = (ss0, ss1)

    def load_idx(k, a):
      pltpu.async_copy(x_hbm.at[pl.ds(row0 + k * IB, IB)],
                       idx_v.at[a], isem[a])

    def wait_idx(a):
      pltpu.make_async_copy(x_hbm.at[pl.ds(0, IB)], idx_v.at[a],
                            isem[a]).wait()

    def gather(a, u, p):
      handles = []
      for i in range(NB):
        for off, n in SPLITS:
          handles.append(pltpu.async_copy(
              table_hbm.at[idx_v.at[a, u * NB + i, pl.ds(off, n)]],
              pair_v.at[p, i, pl.ds(off, n)], gsem[p]))
      for h in handles:
        h.wait()

    def store(s, p):
      pltpu.async_copy(pair_v.at[p],
                       out_hbm.at[pl.ds(row0 + s * NB, NB)], ssem[p])

    def wait_store(p):
      pltpu.make_async_copy(pair_v.at[p],
                            out_hbm.at[pl.ds(0, NB)], ssem[p]).wait()

    def run_chunk(k, a, first):
      wait_idx(a)
      for u in range(NSUBC):
        p = u % 2
        if not (first and u < 2):
          wait_store(p)
        gather(a, u, p)
        store(k * NSUBC + u, p)
      load_idx(jnp.minimum(k + 2, NIDX - 1), a)

    load_idx(0, 0)
    load_idx(1, 1)
    run_chunk(0, 0, True)
    run_chunk(1, 1, False)

    def body(j, carry):
      for a in range(2):
        run_chunk(2 + j * 2 + a, a, False)
      return carry

    lax.fori_loop(0, (NIDX - 2) // 2, body, 0, unroll=False)

    for p in range(2):
      wait_store(p)
      wait_idx(p)

  return sc_gather


_sc_gather = _make_sc_gather()


@jax.jit
def kernel(x, table):
  tp = jnp.pad(table, ((0, 0), (0, PAD - DIM)))
  return _sc_gather(x.astype(jnp.int32), tp)[:, :, :DIM]
